# Initial kernel scaffold; baseline (speedup 1.0000x reference)
import jax
import jax.numpy as jnp
from jax.experimental import pallas as pl


def kernel(x_v, x_c, edge_index, W_in, b_in, Wu_c, bu_c, Wu_v, bu_v, Whm1, bhm1, Whm2, bhm2, Whl1, bhl1, Whl2, bhl2):
    raise NotImplementedError("write your pallas kernel here")


# speedup vs baseline: 6.2795x; 6.2795x over previous
import functools

import jax
import jax.numpy as jnp
from jax import lax
from jax.experimental import pallas as pl
from jax.experimental.pallas import tpu as pltpu
from jax.experimental.pallas import tpu_sc as plsc

N = 5000
NPAD = 5120
H = 128
D = 128
L = 64
E = 320000
NC, NS = 2, 16
NW = NC * NS
CHUNK = 128
E_PAD = 327680
EPW = E_PAD // NW
CPW = EPW // CHUNK
RPT = NPAD // NS
GNN_STEPS = 3

_SC_MESH = plsc.VectorSubcoreMesh(core_axis_name="c", subcore_axis_name="s")


def _sc_messages_body(hc, hv, srcp, dstp, zeros, out,
                      sidx, didx, rowa, rowb, accc, accv, sema, semb):
    cid = lax.axis_index("c")
    sid = lax.axis_index("s")
    wid = cid * NS + sid
    rbase = sid * RPT

    pltpu.sync_copy(zeros, accc.at[pl.ds(rbase, RPT)])
    pltpu.sync_copy(zeros, accv.at[pl.ds(rbase, RPT)])
    plsc.subcore_barrier()

    ebase = wid * EPW

    def chunk_body(ci, carry):
        off = ebase + ci * CHUNK
        pltpu.sync_copy(srcp.at[pl.ds(off, CHUNK)], sidx)
        pltpu.sync_copy(dstp.at[pl.ds(off, CHUNK)], didx)
        ga = pltpu.async_copy(hv.at[sidx], rowa, sema)
        gb = pltpu.async_copy(hc.at[didx], rowb, semb)
        ga.wait()
        pltpu.sync_copy(rowa, accc.at[didx], add=True)
        gb.wait()
        pltpu.sync_copy(rowb, accv.at[sidx], add=True)
        return carry

    lax.fori_loop(0, CPW, chunk_body, 0)
    plsc.subcore_barrier()

    pltpu.sync_copy(accc.at[pl.ds(rbase, RPT)], out.at[cid, 0, pl.ds(rbase, RPT)])
    pltpu.sync_copy(accv.at[pl.ds(rbase, RPT)], out.at[cid, 1, pl.ds(rbase, RPT)])


_sc_messages = functools.partial(
    pl.kernel,
    out_type=jax.ShapeDtypeStruct((NC, 2, NPAD, H), jnp.float32),
    mesh=_SC_MESH,
    scratch_types=[
        pltpu.VMEM((CHUNK,), jnp.int32),
        pltpu.VMEM((CHUNK,), jnp.int32),
        pltpu.VMEM((CHUNK, H), jnp.float32),
        pltpu.VMEM((CHUNK, H), jnp.float32),
        pltpu.VMEM_SHARED((NPAD, H), jnp.float32),
        pltpu.VMEM_SHARED((NPAD, H), jnp.float32),
        pltpu.SemaphoreType.DMA,
        pltpu.SemaphoreType.DMA,
    ],
)(_sc_messages_body)


BU = 640


def _input_body(xc, xv, w, b, hc_out, hv_out):
    wf = w[...]
    bf = b[...]
    hc_out[...] = jnp.maximum(jnp.dot(xc[...], wf,
                                      preferred_element_type=jnp.float32) + bf, 0.0)
    hv_out[...] = jnp.maximum(jnp.dot(xv[...], wf,
                                      preferred_element_type=jnp.float32) + bf, 0.0)


def _tc_input(xc, xv, w, b):
    grid = (NPAD // BU,)
    return pl.pallas_call(
        _input_body,
        grid=grid,
        in_specs=[
            pl.BlockSpec((BU, D), lambda i: (i, 0)),
            pl.BlockSpec((BU, D), lambda i: (i, 0)),
            pl.BlockSpec((D, H), lambda i: (0, 0)),
            pl.BlockSpec((1, H), lambda i: (0, 0)),
        ],
        out_specs=[
            pl.BlockSpec((BU, H), lambda i: (i, 0)),
            pl.BlockSpec((BU, H), lambda i: (i, 0)),
        ],
        out_shape=[
            jax.ShapeDtypeStruct((NPAD, H), jnp.float32),
            jax.ShapeDtypeStruct((NPAD, H), jnp.float32),
        ],
    )(xc, xv, w, b)


def _update_body(hc, hv, mp, wc, wv, bc, bv, hc_out, hv_out):
    mc = mp[0, 0] + mp[1, 0]
    mv = mp[0, 1] + mp[1, 1]
    wcf = wc[...]
    wvf = wv[...]
    hc_out[...] = jnp.maximum(
        jnp.dot(hc[...], wcf[:H], preferred_element_type=jnp.float32)
        + jnp.dot(mc, wcf[H:], preferred_element_type=jnp.float32) + bc[...], 0.0)
    hv_out[...] = jnp.maximum(
        jnp.dot(hv[...], wvf[:H], preferred_element_type=jnp.float32)
        + jnp.dot(mv, wvf[H:], preferred_element_type=jnp.float32) + bv[...], 0.0)


def _tc_update(hc, hv, mpart, wc, wv, bc, bv):
    grid = (NPAD // BU,)
    return pl.pallas_call(
        _update_body,
        grid=grid,
        in_specs=[
            pl.BlockSpec((BU, H), lambda i: (i, 0)),
            pl.BlockSpec((BU, H), lambda i: (i, 0)),
            pl.BlockSpec((NC, 2, BU, H), lambda i: (0, 0, i, 0)),
            pl.BlockSpec((2 * H, H), lambda i: (0, 0)),
            pl.BlockSpec((2 * H, H), lambda i: (0, 0)),
            pl.BlockSpec((1, H), lambda i: (0, 0)),
            pl.BlockSpec((1, H), lambda i: (0, 0)),
        ],
        out_specs=[
            pl.BlockSpec((BU, H), lambda i: (i, 0)),
            pl.BlockSpec((BU, H), lambda i: (i, 0)),
        ],
        out_shape=[
            jax.ShapeDtypeStruct((NPAD, H), jnp.float32),
            jax.ShapeDtypeStruct((NPAD, H), jnp.float32),
        ],
    )(hc, hv, mpart, wc, wv, bc, bv)


def _heads_body(hc, hv, wm1, bm1, wm2, bm2, wl1, bl1, wl2, bl2, ec, ev,
                mean_c, mean_v, lv_c, lv_v, zc, zv):
    wm1f, wm2f, wl1f, wl2f = wm1[...], wm2[...], wl1[...], wl2[...]
    bm1f, bm2f, bl1f, bl2f = bm1[...], bm2[...], bl1[...], bl2[...]

    def head(h, w1, b1, w2, b2):
        t = jnp.dot(h, w1, preferred_element_type=jnp.float32) + b1
        return jnp.dot(t, w2, preferred_element_type=jnp.float32) + b2

    hcf = hc[...]
    hvf = hv[...]
    mc = head(hcf, wm1f, bm1f, wm2f, bm2f)
    mv = head(hvf, wm1f, bm1f, wm2f, bm2f)
    lc = head(hcf, wl1f, bl1f, wl2f, bl2f)
    lv = head(hvf, wl1f, bl1f, wl2f, bl2f)
    mean_c[...] = mc
    mean_v[...] = mv
    lv_c[...] = lc
    lv_v[...] = lv
    zc[...] = mc + ec[...] * jnp.exp(0.5 * lc)
    zv[...] = mv + ev[...] * jnp.exp(0.5 * lv)


def _tc_heads(hc, hv, wm1, bm1, wm2, bm2, wl1, bl1, wl2, bl2, ec, ev):
    grid = (NPAD // BU,)
    blk = lambda r, c: pl.BlockSpec((r, c), lambda i: (i, 0))
    full = lambda r, c: pl.BlockSpec((r, c), lambda i: (0, 0))
    return pl.pallas_call(
        _heads_body,
        grid=grid,
        in_specs=[
            blk(BU, H), blk(BU, H),
            full(H, 32), full(1, 32), full(32, L), full(1, L),
            full(H, 32), full(1, 32), full(32, L), full(1, L),
            blk(BU, L), blk(BU, L),
        ],
        out_specs=[blk(BU, L)] * 6,
        out_shape=[jax.ShapeDtypeStruct((NPAD, L), jnp.float32)] * 6,
    )(hc, hv, wm1, bm1, wm2, bm2, wl1, bl1, wl2, bl2, ec, ev)


BD = 200


def _decoder_body(zv, zc, out):
    logits = jax.lax.dot_general(zv[...], zc[...],
                                 (((1,), (1,)), ((), ())),
                                 preferred_element_type=jnp.float32)
    out[...] = jax.nn.sigmoid(logits)


def _tc_decoder(zv, zc):
    grid = (N // BD,)
    return pl.pallas_call(
        _decoder_body,
        grid=grid,
        in_specs=[
            pl.BlockSpec((BD, L), lambda i: (i, 0)),
            pl.BlockSpec((N, L), lambda i: (0, 0)),
        ],
        out_specs=pl.BlockSpec((BD, N), lambda i: (i, 0)),
        out_shape=jax.ShapeDtypeStruct((N, N), jnp.float32),
    )(zv, zc)


def kernel(x_v, x_c, edge_index, W_in, b_in, Wu_c, bu_c, Wu_v, bu_v,
           Whm1, bhm1, Whm2, bhm2, Whl1, bhl1, Whl2, bhl2):
    f32 = jnp.float32
    pad_rows = NPAD - N

    xv_p = jnp.pad(x_v, ((0, pad_rows), (0, 0)))
    xc_p = jnp.pad(x_c, ((0, pad_rows), (0, 0)))

    pad_e = E_PAD - E
    pad_idx = (N + (jnp.arange(pad_e, dtype=jnp.int32) % pad_rows)).astype(jnp.int32)
    src_p = jnp.concatenate([edge_index[0].astype(jnp.int32), pad_idx])
    dst_p = jnp.concatenate([edge_index[1].astype(jnp.int32), pad_idx])

    zeros_init = jnp.zeros((RPT, H), f32)

    b_in2 = b_in.reshape(1, H)
    bu_c2 = bu_c.reshape(1, H)
    bu_v2 = bu_v.reshape(1, H)

    h_c, h_v = _tc_input(xc_p, xv_p, W_in, b_in2)
    for _ in range(GNN_STEPS):
        mpart = _sc_messages(h_c, h_v, src_p, dst_p, zeros_init)
        h_c, h_v = _tc_update(h_c, h_v, mpart, Wu_c, Wu_v, bu_c2, bu_v2)

    k1, k2 = jax.random.split(jax.random.key(42))
    eps_v = jax.random.normal(k1, (N, L), dtype=f32)
    eps_c = jax.random.normal(k2, (N, L), dtype=f32)
    ev_p = jnp.pad(eps_v, ((0, pad_rows), (0, 0)))
    ec_p = jnp.pad(eps_c, ((0, pad_rows), (0, 0)))

    mean_c, mean_v, lv_c, lv_v, z_c, z_v = _tc_heads(
        h_c, h_v, Whm1, bhm1.reshape(1, 32), Whm2, bhm2.reshape(1, L),
        Whl1, bhl1.reshape(1, 32), Whl2, bhl2.reshape(1, L), ec_p, ev_p)

    adj = _tc_decoder(z_v[:N], z_c[:N])

    return (adj,
            (mean_v[:N], mean_c[:N]),
            (lv_v[:N], lv_c[:N]))


# speedup vs baseline: 7.8708x; 1.2534x over previous
import functools

import jax
import jax.numpy as jnp
from jax import lax
from jax.experimental import pallas as pl
from jax.experimental.pallas import tpu as pltpu
from jax.experimental.pallas import tpu_sc as plsc

N = 5000
NPAD = 5120
H = 128
D = 128
L = 64
E = 320000
NC, NS = 2, 16
NW = NC * NS
CHUNK = 80
E_PAD = 327680
EPW = E_PAD // NW
CPW = EPW // CHUNK
RPT = NPAD // NS
GNN_STEPS = 3

_SC_MESH = plsc.VectorSubcoreMesh(core_axis_name="c", subcore_axis_name="s")


RING = 2
NT = CPW // RING


def _sc_messages_body(hc, hv, srcp, dstp, zeros, out,
                      sidx, didx, rowa, rowb, accc, accv,
                      semi0, semi1, semg0, semg1, sems0, sems1):
    cid = lax.axis_index("c")
    sid = lax.axis_index("s")
    wid = cid * NS + sid
    rbase = sid * RPT

    pltpu.sync_copy(zeros, accc.at[pl.ds(rbase, RPT)])
    pltpu.sync_copy(zeros, accv.at[pl.ds(rbase, RPT)])
    plsc.subcore_barrier()

    ebase = wid * EPW
    semi = (semi0, semi1)
    semg = (semg0, semg1)
    sems = (sems0, sems1)

    def issue_idx(t, h):
        for j in range(RING):
            off = ebase + (t * RING + j) * CHUNK
            pltpu.async_copy(srcp.at[pl.ds(off, CHUNK)], sidx.at[h, j], semi[h])
            pltpu.async_copy(dstp.at[pl.ds(off, CHUNK)], didx.at[h, j], semi[h])

    def wait_idx(h):
        for j in range(RING):
            pltpu.make_async_copy(srcp.at[pl.ds(0, CHUNK)], sidx.at[h, j], semi[h]).wait()
            pltpu.make_async_copy(dstp.at[pl.ds(0, CHUNK)], didx.at[h, j], semi[h]).wait()

    def drain_scatters():
        for j in range(RING):
            pltpu.make_async_copy(hv.at[pl.ds(0, CHUNK)], rowa.at[j], sems[j]).wait()
            pltpu.make_async_copy(hv.at[pl.ds(0, CHUNK)], rowb.at[j], sems[j]).wait()

    issue_idx(0, 0)

    def super_body(t, carry):
        @pl.when(t > 0)
        def _():
            drain_scatters()

        for h in range(2):
            @pl.when(lax.rem(t, 2) == h)
            def _():
                @pl.when(t + 1 < NT)
                def _():
                    issue_idx(t + 1, 1 - h)
                wait_idx(h)
                for j in range(RING):
                    pltpu.async_copy(hv.at[sidx.at[h, j]], rowa.at[j], semg[j])
                    pltpu.async_copy(hc.at[didx.at[h, j]], rowb.at[j], semg[j])
                for j in range(RING):
                    pltpu.make_async_copy(hv.at[pl.ds(0, CHUNK)], rowa.at[j], semg[j]).wait()
                    pltpu.make_async_copy(hv.at[pl.ds(0, CHUNK)], rowb.at[j], semg[j]).wait()
                    pltpu.async_copy(rowa.at[j], accc.at[didx.at[h, j]], sems[j], add=True)
                    pltpu.async_copy(rowb.at[j], accv.at[sidx.at[h, j]], sems[j], add=True)

        return carry

    lax.fori_loop(0, NT, super_body, 0)

    drain_scatters()
    plsc.subcore_barrier()

    pltpu.sync_copy(accc.at[pl.ds(rbase, RPT)], out.at[cid, 0, pl.ds(rbase, RPT)])
    pltpu.sync_copy(accv.at[pl.ds(rbase, RPT)], out.at[cid, 1, pl.ds(rbase, RPT)])


_sc_messages = functools.partial(
    pl.kernel,
    out_type=jax.ShapeDtypeStruct((NC, 2, NPAD, H), jnp.float32),
    mesh=_SC_MESH,
    scratch_types=[
        pltpu.VMEM((2, RING, CHUNK), jnp.int32),
        pltpu.VMEM((2, RING, CHUNK), jnp.int32),
        pltpu.VMEM((RING, CHUNK, H), jnp.float32),
        pltpu.VMEM((RING, CHUNK, H), jnp.float32),
        pltpu.VMEM_SHARED((NPAD, H), jnp.float32),
        pltpu.VMEM_SHARED((NPAD, H), jnp.float32),
    ] + [pltpu.SemaphoreType.DMA] * 6,
)(_sc_messages_body)


BU = 640


def _input_body(xc, xv, w, b, hc_out, hv_out):
    wf = w[...]
    bf = b[...]
    hc_out[...] = jnp.maximum(jnp.dot(xc[...], wf,
                                      preferred_element_type=jnp.float32) + bf, 0.0)
    hv_out[...] = jnp.maximum(jnp.dot(xv[...], wf,
                                      preferred_element_type=jnp.float32) + bf, 0.0)


def _tc_input(xc, xv, w, b):
    grid = (NPAD // BU,)
    return pl.pallas_call(
        _input_body,
        grid=grid,
        in_specs=[
            pl.BlockSpec((BU, D), lambda i: (i, 0)),
            pl.BlockSpec((BU, D), lambda i: (i, 0)),
            pl.BlockSpec((D, H), lambda i: (0, 0)),
            pl.BlockSpec((1, H), lambda i: (0, 0)),
        ],
        out_specs=[
            pl.BlockSpec((BU, H), lambda i: (i, 0)),
            pl.BlockSpec((BU, H), lambda i: (i, 0)),
        ],
        out_shape=[
            jax.ShapeDtypeStruct((NPAD, H), jnp.float32),
            jax.ShapeDtypeStruct((NPAD, H), jnp.float32),
        ],
    )(xc, xv, w, b)


def _update_body(hc, hv, mp, wc, wv, bc, bv, hc_out, hv_out):
    mc = mp[0, 0] + mp[1, 0]
    mv = mp[0, 1] + mp[1, 1]
    wcf = wc[...]
    wvf = wv[...]
    hc_out[...] = jnp.maximum(
        jnp.dot(hc[...], wcf[:H], preferred_element_type=jnp.float32)
        + jnp.dot(mc, wcf[H:], preferred_element_type=jnp.float32) + bc[...], 0.0)
    hv_out[...] = jnp.maximum(
        jnp.dot(hv[...], wvf[:H], preferred_element_type=jnp.float32)
        + jnp.dot(mv, wvf[H:], preferred_element_type=jnp.float32) + bv[...], 0.0)


def _tc_update(hc, hv, mpart, wc, wv, bc, bv):
    grid = (NPAD // BU,)
    return pl.pallas_call(
        _update_body,
        grid=grid,
        in_specs=[
            pl.BlockSpec((BU, H), lambda i: (i, 0)),
            pl.BlockSpec((BU, H), lambda i: (i, 0)),
            pl.BlockSpec((NC, 2, BU, H), lambda i: (0, 0, i, 0)),
            pl.BlockSpec((2 * H, H), lambda i: (0, 0)),
            pl.BlockSpec((2 * H, H), lambda i: (0, 0)),
            pl.BlockSpec((1, H), lambda i: (0, 0)),
            pl.BlockSpec((1, H), lambda i: (0, 0)),
        ],
        out_specs=[
            pl.BlockSpec((BU, H), lambda i: (i, 0)),
            pl.BlockSpec((BU, H), lambda i: (i, 0)),
        ],
        out_shape=[
            jax.ShapeDtypeStruct((NPAD, H), jnp.float32),
            jax.ShapeDtypeStruct((NPAD, H), jnp.float32),
        ],
    )(hc, hv, mpart, wc, wv, bc, bv)


def _heads_body(hc, hv, wm1, bm1, wm2, bm2, wl1, bl1, wl2, bl2, ec, ev,
                mean_c, mean_v, lv_c, lv_v, zc, zv):
    wm1f, wm2f, wl1f, wl2f = wm1[...], wm2[...], wl1[...], wl2[...]
    bm1f, bm2f, bl1f, bl2f = bm1[...], bm2[...], bl1[...], bl2[...]

    def head(h, w1, b1, w2, b2):
        t = jnp.dot(h, w1, preferred_element_type=jnp.float32) + b1
        return jnp.dot(t, w2, preferred_element_type=jnp.float32) + b2

    hcf = hc[...]
    hvf = hv[...]
    mc = head(hcf, wm1f, bm1f, wm2f, bm2f)
    mv = head(hvf, wm1f, bm1f, wm2f, bm2f)
    lc = head(hcf, wl1f, bl1f, wl2f, bl2f)
    lv = head(hvf, wl1f, bl1f, wl2f, bl2f)
    mean_c[...] = mc
    mean_v[...] = mv
    lv_c[...] = lc
    lv_v[...] = lv
    zc[...] = mc + ec[...] * jnp.exp(0.5 * lc)
    zv[...] = mv + ev[...] * jnp.exp(0.5 * lv)


def _tc_heads(hc, hv, wm1, bm1, wm2, bm2, wl1, bl1, wl2, bl2, ec, ev):
    grid = (NPAD // BU,)
    blk = lambda r, c: pl.BlockSpec((r, c), lambda i: (i, 0))
    full = lambda r, c: pl.BlockSpec((r, c), lambda i: (0, 0))
    return pl.pallas_call(
        _heads_body,
        grid=grid,
        in_specs=[
            blk(BU, H), blk(BU, H),
            full(H, 32), full(1, 32), full(32, L), full(1, L),
            full(H, 32), full(1, 32), full(32, L), full(1, L),
            blk(BU, L), blk(BU, L),
        ],
        out_specs=[blk(BU, L)] * 6,
        out_shape=[jax.ShapeDtypeStruct((NPAD, L), jnp.float32)] * 6,
    )(hc, hv, wm1, bm1, wm2, bm2, wl1, bl1, wl2, bl2, ec, ev)


BD = 200


def _decoder_body(zv, zc, out):
    logits = jax.lax.dot_general(zv[...], zc[...],
                                 (((1,), (1,)), ((), ())),
                                 preferred_element_type=jnp.float32)
    out[...] = jax.nn.sigmoid(logits)


def _tc_decoder(zv, zc):
    grid = (N // BD,)
    return pl.pallas_call(
        _decoder_body,
        grid=grid,
        in_specs=[
            pl.BlockSpec((BD, L), lambda i: (i, 0)),
            pl.BlockSpec((N, L), lambda i: (0, 0)),
        ],
        out_specs=pl.BlockSpec((BD, N), lambda i: (i, 0)),
        out_shape=jax.ShapeDtypeStruct((N, N), jnp.float32),
    )(zv, zc)


def kernel(x_v, x_c, edge_index, W_in, b_in, Wu_c, bu_c, Wu_v, bu_v,
           Whm1, bhm1, Whm2, bhm2, Whl1, bhl1, Whl2, bhl2):
    f32 = jnp.float32
    pad_rows = NPAD - N

    xv_p = jnp.pad(x_v, ((0, pad_rows), (0, 0)))
    xc_p = jnp.pad(x_c, ((0, pad_rows), (0, 0)))

    pad_e = E_PAD - E
    pad_idx = (N + (jnp.arange(pad_e, dtype=jnp.int32) % pad_rows)).astype(jnp.int32)
    src_p = jnp.concatenate([edge_index[0].astype(jnp.int32), pad_idx])
    dst_p = jnp.concatenate([edge_index[1].astype(jnp.int32), pad_idx])

    zeros_init = jnp.zeros((RPT, H), f32)

    b_in2 = b_in.reshape(1, H)
    bu_c2 = bu_c.reshape(1, H)
    bu_v2 = bu_v.reshape(1, H)

    h_c, h_v = _tc_input(xc_p, xv_p, W_in, b_in2)
    for _ in range(GNN_STEPS):
        mpart = _sc_messages(h_c, h_v, src_p, dst_p, zeros_init)
        h_c, h_v = _tc_update(h_c, h_v, mpart, Wu_c, Wu_v, bu_c2, bu_v2)

    k1, k2 = jax.random.split(jax.random.key(42))
    eps_v = jax.random.normal(k1, (N, L), dtype=f32)
    eps_c = jax.random.normal(k2, (N, L), dtype=f32)
    ev_p = jnp.pad(eps_v, ((0, pad_rows), (0, 0)))
    ec_p = jnp.pad(eps_c, ((0, pad_rows), (0, 0)))

    mean_c, mean_v, lv_c, lv_v, z_c, z_v = _tc_heads(
        h_c, h_v, Whm1, bhm1.reshape(1, 32), Whm2, bhm2.reshape(1, L),
        Whl1, bhl1.reshape(1, 32), Whl2, bhl2.reshape(1, L), ec_p, ev_p)

    adj = _tc_decoder(z_v[:N], z_c[:N])

    return (adj,
            (mean_v[:N], mean_c[:N]),
            (lv_v[:N], lv_c[:N]))


# speedup vs baseline: 8.2460x; 1.0477x over previous
import functools

import jax
import jax.numpy as jnp
from jax import lax
from jax.experimental import pallas as pl
from jax.experimental.pallas import tpu as pltpu
from jax.experimental.pallas import tpu_sc as plsc

N = 5000
NPAD = 5120
H = 128
D = 128
L = 64
E = 320000
NC, NS = 2, 16
NW = NC * NS
CHUNK = 128
E_PAD = 327680
EPT = E_PAD // NS
CPT = EPT // CHUNK
RPT = NPAD // NS
GNN_STEPS = 3

_SC_MESH = plsc.VectorSubcoreMesh(core_axis_name="c", subcore_axis_name="s")


RING = 4
NT = CPT // RING


def _sc_messages_body(hc, hv, idxp0, idxp1, zeros, out,
                      idxb, rows, acc,
                      semi0, semi1,
                      semg0, semg1, semg2, semg3,
                      sems0, sems1, sems2, sems3):
    cid = lax.axis_index("c")
    sid = lax.axis_index("s")
    rbase = sid * RPT

    semi = (semi0, semi1)
    semg = (semg0, semg1, semg2, semg3)
    sems = (sems0, sems1, sems2, sems3)

    def run_direction(table, idxp, d):
        pltpu.sync_copy(zeros, acc.at[pl.ds(rbase, RPT)])
        plsc.subcore_barrier()
        cbase = sid * CPT

        def issue_idx(t, h):
            for j in range(RING):
                g = cbase + t * RING + j
                pltpu.async_copy(idxp.at[g], idxb.at[h, j], semi[h])

        def wait_idx(h):
            for j in range(RING):
                pltpu.make_async_copy(idxp.at[0], idxb.at[h, j], semi[h]).wait()

        def drain_scatters():
            for j in range(RING):
                pltpu.make_async_copy(table.at[pl.ds(0, CHUNK)], rows.at[j],
                                      sems[j]).wait()

        issue_idx(0, 0)

        def super_body(t, carry):
            @pl.when(t > 0)
            def _():
                drain_scatters()

            for h in range(2):
                @pl.when(lax.rem(t, 2) == h)
                def _():
                    @pl.when(t + 1 < NT)
                    def _():
                        issue_idx(t + 1, 1 - h)
                    wait_idx(h)
                    for j in range(RING):
                        pltpu.async_copy(table.at[idxb.at[h, j, 0]], rows.at[j],
                                         semg[j])
                    for j in range(RING):
                        pltpu.make_async_copy(table.at[pl.ds(0, CHUNK)],
                                              rows.at[j], semg[j]).wait()
                        pltpu.async_copy(rows.at[j], acc.at[idxb.at[h, j, 1]],
                                         sems[j], add=True)
            return carry

        lax.fori_loop(0, NT, super_body, 0)
        drain_scatters()
        plsc.subcore_barrier()
        pltpu.sync_copy(acc.at[pl.ds(rbase, RPT)], out.at[d, pl.ds(rbase, RPT)])

    @pl.when(cid == 0)
    def _():
        run_direction(hv, idxp0, 0)

    @pl.when(cid == 1)
    def _():
        run_direction(hc, idxp1, 1)


_sc_messages = functools.partial(
    pl.kernel,
    out_type=jax.ShapeDtypeStruct((2, NPAD, H), jnp.float32),
    mesh=_SC_MESH,
    scratch_types=[
        pltpu.VMEM((2, RING, 2, CHUNK), jnp.int32),
        pltpu.VMEM((RING, CHUNK, H), jnp.float32),
        pltpu.VMEM_SHARED((NPAD, H), jnp.float32),
    ] + [pltpu.SemaphoreType.DMA] * 10,
)(_sc_messages_body)


BU = 640


def _input_body(xc, xv, w, b, hc_out, hv_out):
    wf = w[...]
    bf = b[...]
    hc_out[...] = jnp.maximum(jnp.dot(xc[...], wf,
                                      preferred_element_type=jnp.float32) + bf, 0.0)
    hv_out[...] = jnp.maximum(jnp.dot(xv[...], wf,
                                      preferred_element_type=jnp.float32) + bf, 0.0)


def _tc_input(xc, xv, w, b):
    grid = (NPAD // BU,)
    return pl.pallas_call(
        _input_body,
        grid=grid,
        in_specs=[
            pl.BlockSpec((BU, D), lambda i: (i, 0)),
            pl.BlockSpec((BU, D), lambda i: (i, 0)),
            pl.BlockSpec((D, H), lambda i: (0, 0)),
            pl.BlockSpec((1, H), lambda i: (0, 0)),
        ],
        out_specs=[
            pl.BlockSpec((BU, H), lambda i: (i, 0)),
            pl.BlockSpec((BU, H), lambda i: (i, 0)),
        ],
        out_shape=[
            jax.ShapeDtypeStruct((NPAD, H), jnp.float32),
            jax.ShapeDtypeStruct((NPAD, H), jnp.float32),
        ],
    )(xc, xv, w, b)


def _update_body(hc, hv, mp, wc, wv, bc, bv, hc_out, hv_out):
    mc = mp[0]
    mv = mp[1]
    wcf = wc[...]
    wvf = wv[...]
    hc_out[...] = jnp.maximum(
        jnp.dot(hc[...], wcf[:H], preferred_element_type=jnp.float32)
        + jnp.dot(mc, wcf[H:], preferred_element_type=jnp.float32) + bc[...], 0.0)
    hv_out[...] = jnp.maximum(
        jnp.dot(hv[...], wvf[:H], preferred_element_type=jnp.float32)
        + jnp.dot(mv, wvf[H:], preferred_element_type=jnp.float32) + bv[...], 0.0)


def _tc_update(hc, hv, mpart, wc, wv, bc, bv):
    grid = (NPAD // BU,)
    return pl.pallas_call(
        _update_body,
        grid=grid,
        in_specs=[
            pl.BlockSpec((BU, H), lambda i: (i, 0)),
            pl.BlockSpec((BU, H), lambda i: (i, 0)),
            pl.BlockSpec((2, BU, H), lambda i: (0, i, 0)),
            pl.BlockSpec((2 * H, H), lambda i: (0, 0)),
            pl.BlockSpec((2 * H, H), lambda i: (0, 0)),
            pl.BlockSpec((1, H), lambda i: (0, 0)),
            pl.BlockSpec((1, H), lambda i: (0, 0)),
        ],
        out_specs=[
            pl.BlockSpec((BU, H), lambda i: (i, 0)),
            pl.BlockSpec((BU, H), lambda i: (i, 0)),
        ],
        out_shape=[
            jax.ShapeDtypeStruct((NPAD, H), jnp.float32),
            jax.ShapeDtypeStruct((NPAD, H), jnp.float32),
        ],
    )(hc, hv, mpart, wc, wv, bc, bv)


def _heads_body(hc, hv, wm1, bm1, wm2, bm2, wl1, bl1, wl2, bl2, ec, ev,
                mean_c, mean_v, lv_c, lv_v, zc, zv):
    wm1f, wm2f, wl1f, wl2f = wm1[...], wm2[...], wl1[...], wl2[...]
    bm1f, bm2f, bl1f, bl2f = bm1[...], bm2[...], bl1[...], bl2[...]

    def head(h, w1, b1, w2, b2):
        t = jnp.dot(h, w1, preferred_element_type=jnp.float32) + b1
        return jnp.dot(t, w2, preferred_element_type=jnp.float32) + b2

    hcf = hc[...]
    hvf = hv[...]
    mc = head(hcf, wm1f, bm1f, wm2f, bm2f)
    mv = head(hvf, wm1f, bm1f, wm2f, bm2f)
    lc = head(hcf, wl1f, bl1f, wl2f, bl2f)
    lv = head(hvf, wl1f, bl1f, wl2f, bl2f)
    mean_c[...] = mc
    mean_v[...] = mv
    lv_c[...] = lc
    lv_v[...] = lv
    zc[...] = mc + ec[...] * jnp.exp(0.5 * lc)
    zv[...] = mv + ev[...] * jnp.exp(0.5 * lv)


def _tc_heads(hc, hv, wm1, bm1, wm2, bm2, wl1, bl1, wl2, bl2, ec, ev):
    grid = (NPAD // BU,)
    blk = lambda r, c: pl.BlockSpec((r, c), lambda i: (i, 0))
    full = lambda r, c: pl.BlockSpec((r, c), lambda i: (0, 0))
    return pl.pallas_call(
        _heads_body,
        grid=grid,
        in_specs=[
            blk(BU, H), blk(BU, H),
            full(H, 32), full(1, 32), full(32, L), full(1, L),
            full(H, 32), full(1, 32), full(32, L), full(1, L),
            blk(BU, L), blk(BU, L),
        ],
        out_specs=[blk(BU, L)] * 6,
        out_shape=[jax.ShapeDtypeStruct((NPAD, L), jnp.float32)] * 6,
    )(hc, hv, wm1, bm1, wm2, bm2, wl1, bl1, wl2, bl2, ec, ev)


BD = 200


def _decoder_body(zv, zc, out):
    logits = jax.lax.dot_general(zv[...], zc[...],
                                 (((1,), (1,)), ((), ())),
                                 preferred_element_type=jnp.float32)
    out[...] = jax.nn.sigmoid(logits)


def _tc_decoder(zv, zc):
    grid = (N // BD,)
    return pl.pallas_call(
        _decoder_body,
        grid=grid,
        in_specs=[
            pl.BlockSpec((BD, L), lambda i: (i, 0)),
            pl.BlockSpec((N, L), lambda i: (0, 0)),
        ],
        out_specs=pl.BlockSpec((BD, N), lambda i: (i, 0)),
        out_shape=jax.ShapeDtypeStruct((N, N), jnp.float32),
    )(zv, zc)


def kernel(x_v, x_c, edge_index, W_in, b_in, Wu_c, bu_c, Wu_v, bu_v,
           Whm1, bhm1, Whm2, bhm2, Whl1, bhl1, Whl2, bhl2):
    f32 = jnp.float32
    pad_rows = NPAD - N

    xv_p = jnp.pad(x_v, ((0, pad_rows), (0, 0)))
    xc_p = jnp.pad(x_c, ((0, pad_rows), (0, 0)))

    pad_e = E_PAD - E
    pad_idx = (N + (jnp.arange(pad_e, dtype=jnp.int32) % pad_rows)).astype(jnp.int32)
    src_p = jnp.concatenate([edge_index[0].astype(jnp.int32), pad_idx]).reshape(-1, CHUNK)
    dst_p = jnp.concatenate([edge_index[1].astype(jnp.int32), pad_idx]).reshape(-1, CHUNK)
    idxp0 = jnp.stack([src_p, dst_p], axis=1)
    idxp1 = jnp.stack([dst_p, src_p], axis=1)

    zeros_init = jnp.zeros((RPT, H), f32)

    b_in2 = b_in.reshape(1, H)
    bu_c2 = bu_c.reshape(1, H)
    bu_v2 = bu_v.reshape(1, H)

    h_c, h_v = _tc_input(xc_p, xv_p, W_in, b_in2)
    for _ in range(GNN_STEPS):
        mpart = _sc_messages(h_c, h_v, idxp0, idxp1, zeros_init)
        h_c, h_v = _tc_update(h_c, h_v, mpart, Wu_c, Wu_v, bu_c2, bu_v2)

    k1, k2 = jax.random.split(jax.random.key(42))
    eps_v = jax.random.normal(k1, (N, L), dtype=f32)
    eps_c = jax.random.normal(k2, (N, L), dtype=f32)
    ev_p = jnp.pad(eps_v, ((0, pad_rows), (0, 0)))
    ec_p = jnp.pad(eps_c, ((0, pad_rows), (0, 0)))

    mean_c, mean_v, lv_c, lv_v, z_c, z_v = _tc_heads(
        h_c, h_v, Whm1, bhm1.reshape(1, 32), Whm2, bhm2.reshape(1, L),
        Whl1, bhl1.reshape(1, 32), Whl2, bhl2.reshape(1, L), ec_p, ev_p)

    adj = _tc_decoder(z_v[:N], z_c[:N])

    return (adj,
            (mean_v[:N], mean_c[:N]),
            (lv_v[:N], lv_c[:N]))


# speedup vs baseline: 10.1933x; 1.2361x over previous
import functools

import jax
import jax.numpy as jnp
from jax import lax
from jax.experimental import pallas as pl
from jax.experimental.pallas import tpu as pltpu
from jax.experimental.pallas import tpu_sc as plsc

N = 5000
NPAD = 5120
H = 128
D = 128
L = 64
E = 320000
NC, NS = 2, 16
NW = NC * NS
CHUNK = 128
E_PAD = 327680
EPT = E_PAD // NS
CPT = EPT // CHUNK
RPT = NPAD // NS
GNN_STEPS = 3

_SC_MESH = plsc.VectorSubcoreMesh(core_axis_name="c", subcore_axis_name="s")


RING = 4
NT = CPT // RING


def _sc_messages_body(hc, hv, idxp0, idxp1, zeros, out,
                      idxb, rows, acc,
                      semi0, semi1,
                      semg0, semg1, semg2, semg3,
                      sems0, sems1, sems2, sems3):
    cid = lax.axis_index("c")
    sid = lax.axis_index("s")
    rbase = sid * RPT

    semi = (semi0, semi1)
    semg = (semg0, semg1, semg2, semg3)
    sems = (sems0, sems1, sems2, sems3)

    def run_direction(table, idxp, d):
        pltpu.sync_copy(zeros, acc.at[pl.ds(rbase, RPT)])
        plsc.subcore_barrier()
        cbase = sid * CPT

        def issue_idx(t, h):
            for j in range(RING):
                g = cbase + t * RING + j
                pltpu.async_copy(idxp.at[g], idxb.at[h, j], semi[h])

        def wait_idx(h):
            for j in range(RING):
                pltpu.make_async_copy(idxp.at[0], idxb.at[h, j], semi[h]).wait()

        def drain_scatters():
            for j in range(RING):
                pltpu.make_async_copy(table.at[pl.ds(0, CHUNK)], rows.at[j],
                                      sems[j]).wait()

        issue_idx(0, 0)

        def super_body(t, carry):
            for h in range(2):
                @pl.when(lax.rem(t, 2) == h)
                def _():
                    wait_idx(h)
                    for j in range(RING):
                        @pl.when(t > 0)
                        def _():
                            pltpu.make_async_copy(table.at[pl.ds(0, CHUNK)],
                                                  rows.at[j], sems[j]).wait()
                        pltpu.async_copy(table.at[idxb.at[h, j, 0]], rows.at[j],
                                         semg[j])
                    @pl.when(t + 1 < NT)
                    def _():
                        issue_idx(t + 1, 1 - h)
                    for j in range(RING):
                        pltpu.make_async_copy(table.at[pl.ds(0, CHUNK)],
                                              rows.at[j], semg[j]).wait()
                        pltpu.async_copy(rows.at[j], acc.at[idxb.at[h, j, 1]],
                                         sems[j], add=True)
            return carry

        lax.fori_loop(0, NT, super_body, 0)
        drain_scatters()
        plsc.subcore_barrier()
        pltpu.sync_copy(acc.at[pl.ds(rbase, RPT)], out.at[d, pl.ds(rbase, RPT)])

    @pl.when(cid == 0)
    def _():
        run_direction(hv, idxp0, 0)

    @pl.when(cid == 1)
    def _():
        run_direction(hc, idxp1, 1)


_sc_messages = functools.partial(
    pl.kernel,
    out_type=jax.ShapeDtypeStruct((2, NPAD, H), jnp.float32),
    mesh=_SC_MESH,
    scratch_types=[
        pltpu.VMEM((2, RING, 2, CHUNK), jnp.int32),
        pltpu.VMEM((RING, CHUNK, H), jnp.float32),
        pltpu.VMEM_SHARED((NPAD, H), jnp.float32),
    ] + [pltpu.SemaphoreType.DMA] * 10,
)(_sc_messages_body)


BU = 640


def _input_body(xc, xv, w, b, hc_out, hv_out):
    wf = w[...]
    bf = b[...]
    hc_out[...] = jnp.maximum(jnp.dot(xc[...], wf,
                                      preferred_element_type=jnp.float32) + bf, 0.0)
    hv_out[...] = jnp.maximum(jnp.dot(xv[...], wf,
                                      preferred_element_type=jnp.float32) + bf, 0.0)


def _tc_input(xc, xv, w, b):
    grid = (NPAD // BU,)
    return pl.pallas_call(
        _input_body,
        grid=grid,
        in_specs=[
            pl.BlockSpec((BU, D), lambda i: (i, 0)),
            pl.BlockSpec((BU, D), lambda i: (i, 0)),
            pl.BlockSpec((D, H), lambda i: (0, 0)),
            pl.BlockSpec((1, H), lambda i: (0, 0)),
        ],
        out_specs=[
            pl.BlockSpec((BU, H), lambda i: (i, 0)),
            pl.BlockSpec((BU, H), lambda i: (i, 0)),
        ],
        out_shape=[
            jax.ShapeDtypeStruct((NPAD, H), jnp.float32),
            jax.ShapeDtypeStruct((NPAD, H), jnp.float32),
        ],
    )(xc, xv, w, b)


def _update_body(hc, hv, mp, wc, wv, bc, bv, hc_out, hv_out):
    mc = mp[0]
    mv = mp[1]
    wcf = wc[...]
    wvf = wv[...]
    hc_out[...] = jnp.maximum(
        jnp.dot(hc[...], wcf[:H], preferred_element_type=jnp.float32)
        + jnp.dot(mc, wcf[H:], preferred_element_type=jnp.float32) + bc[...], 0.0)
    hv_out[...] = jnp.maximum(
        jnp.dot(hv[...], wvf[:H], preferred_element_type=jnp.float32)
        + jnp.dot(mv, wvf[H:], preferred_element_type=jnp.float32) + bv[...], 0.0)


def _tc_update(hc, hv, mpart, wc, wv, bc, bv):
    grid = (NPAD // BU,)
    return pl.pallas_call(
        _update_body,
        grid=grid,
        in_specs=[
            pl.BlockSpec((BU, H), lambda i: (i, 0)),
            pl.BlockSpec((BU, H), lambda i: (i, 0)),
            pl.BlockSpec((2, BU, H), lambda i: (0, i, 0)),
            pl.BlockSpec((2 * H, H), lambda i: (0, 0)),
            pl.BlockSpec((2 * H, H), lambda i: (0, 0)),
            pl.BlockSpec((1, H), lambda i: (0, 0)),
            pl.BlockSpec((1, H), lambda i: (0, 0)),
        ],
        out_specs=[
            pl.BlockSpec((BU, H), lambda i: (i, 0)),
            pl.BlockSpec((BU, H), lambda i: (i, 0)),
        ],
        out_shape=[
            jax.ShapeDtypeStruct((NPAD, H), jnp.float32),
            jax.ShapeDtypeStruct((NPAD, H), jnp.float32),
        ],
    )(hc, hv, mpart, wc, wv, bc, bv)


def _heads_body(hc, hv, wm1, bm1, wm2, bm2, wl1, bl1, wl2, bl2, ec, ev,
                mean_c, mean_v, lv_c, lv_v, zc, zv):
    wm1f, wm2f, wl1f, wl2f = wm1[...], wm2[...], wl1[...], wl2[...]
    bm1f, bm2f, bl1f, bl2f = bm1[...], bm2[...], bl1[...], bl2[...]

    def head(h, w1, b1, w2, b2):
        t = jnp.dot(h, w1, preferred_element_type=jnp.float32) + b1
        return jnp.dot(t, w2, preferred_element_type=jnp.float32) + b2

    hcf = hc[...]
    hvf = hv[...]
    mc = head(hcf, wm1f, bm1f, wm2f, bm2f)
    mv = head(hvf, wm1f, bm1f, wm2f, bm2f)
    lc = head(hcf, wl1f, bl1f, wl2f, bl2f)
    lv = head(hvf, wl1f, bl1f, wl2f, bl2f)
    mean_c[...] = mc
    mean_v[...] = mv
    lv_c[...] = lc
    lv_v[...] = lv
    zc[...] = mc + ec[...] * jnp.exp(0.5 * lc)
    zv[...] = mv + ev[...] * jnp.exp(0.5 * lv)


def _tc_heads(hc, hv, wm1, bm1, wm2, bm2, wl1, bl1, wl2, bl2, ec, ev):
    grid = (NPAD // BU,)
    blk = lambda r, c: pl.BlockSpec((r, c), lambda i: (i, 0))
    full = lambda r, c: pl.BlockSpec((r, c), lambda i: (0, 0))
    return pl.pallas_call(
        _heads_body,
        grid=grid,
        in_specs=[
            blk(BU, H), blk(BU, H),
            full(H, 32), full(1, 32), full(32, L), full(1, L),
            full(H, 32), full(1, 32), full(32, L), full(1, L),
            blk(BU, L), blk(BU, L),
        ],
        out_specs=[blk(BU, L)] * 6,
        out_shape=[jax.ShapeDtypeStruct((NPAD, L), jnp.float32)] * 6,
    )(hc, hv, wm1, bm1, wm2, bm2, wl1, bl1, wl2, bl2, ec, ev)


BD = 200


def _decoder_body(zv, zc, out):
    logits = jax.lax.dot_general(zv[...], zc[...],
                                 (((1,), (1,)), ((), ())),
                                 preferred_element_type=jnp.float32)
    out[...] = jax.nn.sigmoid(logits)


def _tc_decoder(zv, zc):
    grid = (N // BD,)
    return pl.pallas_call(
        _decoder_body,
        grid=grid,
        in_specs=[
            pl.BlockSpec((BD, L), lambda i: (i, 0)),
            pl.BlockSpec((N, L), lambda i: (0, 0)),
        ],
        out_specs=pl.BlockSpec((BD, N), lambda i: (i, 0)),
        out_shape=jax.ShapeDtypeStruct((N, N), jnp.float32),
    )(zv, zc)


def kernel(x_v, x_c, edge_index, W_in, b_in, Wu_c, bu_c, Wu_v, bu_v,
           Whm1, bhm1, Whm2, bhm2, Whl1, bhl1, Whl2, bhl2):
    f32 = jnp.float32
    pad_rows = NPAD - N

    xv_p = jnp.pad(x_v, ((0, pad_rows), (0, 0)))
    xc_p = jnp.pad(x_c, ((0, pad_rows), (0, 0)))

    pad_e = E_PAD - E
    pad_idx = (N + (jnp.arange(pad_e, dtype=jnp.int32) % pad_rows)).astype(jnp.int32)
    src_p = jnp.concatenate([edge_index[0].astype(jnp.int32), pad_idx]).reshape(-1, CHUNK)
    dst_p = jnp.concatenate([edge_index[1].astype(jnp.int32), pad_idx]).reshape(-1, CHUNK)
    idxp0 = jnp.stack([src_p, dst_p], axis=1)
    idxp1 = jnp.stack([dst_p, src_p], axis=1)

    zeros_init = jnp.zeros((RPT, H), f32)

    b_in2 = b_in.reshape(1, H)
    bu_c2 = bu_c.reshape(1, H)
    bu_v2 = bu_v.reshape(1, H)

    h_c, h_v = _tc_input(xc_p, xv_p, W_in, b_in2)
    for _ in range(GNN_STEPS):
        mpart = _sc_messages(h_c, h_v, idxp0, idxp1, zeros_init)
        h_c, h_v = _tc_update(h_c, h_v, mpart, Wu_c, Wu_v, bu_c2, bu_v2)

    k1, k2 = jax.random.split(jax.random.key(42))
    eps_v = jax.random.normal(k1, (N, L), dtype=f32)
    eps_c = jax.random.normal(k2, (N, L), dtype=f32)
    ev_p = jnp.pad(eps_v, ((0, pad_rows), (0, 0)))
    ec_p = jnp.pad(eps_c, ((0, pad_rows), (0, 0)))

    mean_c, mean_v, lv_c, lv_v, z_c, z_v = _tc_heads(
        h_c, h_v, Whm1, bhm1.reshape(1, 32), Whm2, bhm2.reshape(1, L),
        Whl1, bhl1.reshape(1, 32), Whl2, bhl2.reshape(1, L), ec_p, ev_p)

    adj = _tc_decoder(z_v[:N], z_c[:N])

    return (adj,
            (mean_v[:N], mean_c[:N]),
            (lv_v[:N], lv_c[:N]))


# speedup vs baseline: 10.2832x; 1.0088x over previous
import functools

import jax
import jax.numpy as jnp
from jax import lax
from jax.experimental import pallas as pl
from jax.experimental.pallas import tpu as pltpu
from jax.experimental.pallas import tpu_sc as plsc

N = 5000
NPAD = 5120
H = 128
D = 128
L = 64
E = 320000
NC, NS = 2, 16
NW = NC * NS
CHUNK = 128
E_PAD = 327680
EPT = E_PAD // NS
CPT = EPT // CHUNK
RPT = NPAD // NS
GNN_STEPS = 3

_SC_MESH = plsc.VectorSubcoreMesh(core_axis_name="c", subcore_axis_name="s")


RING = 5
NT = CPT // RING


def _sc_messages_body(hc, hv, idxp0, idxp1, zeros, out,
                      idxb, rows, acc,
                      semi0, semi1,
                      semg0, semg1, semg2, semg3, semg4,
                      sems0, sems1, sems2, sems3, sems4):
    cid = lax.axis_index("c")
    sid = lax.axis_index("s")
    rbase = sid * RPT

    semi = (semi0, semi1)
    semg = (semg0, semg1, semg2, semg3, semg4)
    sems = (sems0, sems1, sems2, sems3, sems4)

    def run_direction(table, idxp, d):
        pltpu.sync_copy(zeros, acc.at[pl.ds(rbase, RPT)])
        plsc.subcore_barrier()
        cbase = sid * CPT

        def issue_idx(t, h):
            for j in range(RING):
                g = cbase + t * RING + j
                pltpu.async_copy(idxp.at[g], idxb.at[h, j], semi[h])

        def wait_idx(h):
            for j in range(RING):
                pltpu.make_async_copy(idxp.at[0], idxb.at[h, j], semi[h]).wait()

        def drain_scatters():
            for j in range(RING):
                pltpu.make_async_copy(table.at[pl.ds(0, CHUNK)], rows.at[j],
                                      sems[j]).wait()

        issue_idx(0, 0)

        def super_body(t, carry):
            for h in range(2):
                @pl.when(lax.rem(t, 2) == h)
                def _():
                    wait_idx(h)
                    for j in range(RING):
                        @pl.when(t > 0)
                        def _():
                            pltpu.make_async_copy(table.at[pl.ds(0, CHUNK)],
                                                  rows.at[j], sems[j]).wait()
                        pltpu.async_copy(table.at[idxb.at[h, j, 0]], rows.at[j],
                                         semg[j])
                    @pl.when(t + 1 < NT)
                    def _():
                        issue_idx(t + 1, 1 - h)
                    for j in range(RING):
                        pltpu.make_async_copy(table.at[pl.ds(0, CHUNK)],
                                              rows.at[j], semg[j]).wait()
                        pltpu.async_copy(rows.at[j], acc.at[idxb.at[h, j, 1]],
                                         sems[j], add=True)
            return carry

        lax.fori_loop(0, NT, super_body, 0)
        drain_scatters()
        plsc.subcore_barrier()
        pltpu.sync_copy(acc.at[pl.ds(rbase, RPT)], out.at[d, pl.ds(rbase, RPT)])

    @pl.when(cid == 0)
    def _():
        run_direction(hv, idxp0, 0)

    @pl.when(cid == 1)
    def _():
        run_direction(hc, idxp1, 1)


_sc_messages = functools.partial(
    pl.kernel,
    out_type=jax.ShapeDtypeStruct((2, NPAD, H), jnp.float32),
    mesh=_SC_MESH,
    scratch_types=[
        pltpu.VMEM((2, RING, 2, CHUNK), jnp.int32),
        pltpu.VMEM((RING, CHUNK, H), jnp.float32),
        pltpu.VMEM_SHARED((NPAD, H), jnp.float32),
    ] + [pltpu.SemaphoreType.DMA] * 12,
)(_sc_messages_body)


BU = 640


def _input_body(xc, xv, w, b, hc_out, hv_out):
    wf = w[...]
    bf = b[...]
    hc_out[...] = jnp.maximum(jnp.dot(xc[...], wf,
                                      preferred_element_type=jnp.float32) + bf, 0.0)
    hv_out[...] = jnp.maximum(jnp.dot(xv[...], wf,
                                      preferred_element_type=jnp.float32) + bf, 0.0)


def _tc_input(xc, xv, w, b):
    grid = (NPAD // BU,)
    return pl.pallas_call(
        _input_body,
        grid=grid,
        in_specs=[
            pl.BlockSpec((BU, D), lambda i: (i, 0)),
            pl.BlockSpec((BU, D), lambda i: (i, 0)),
            pl.BlockSpec((D, H), lambda i: (0, 0)),
            pl.BlockSpec((1, H), lambda i: (0, 0)),
        ],
        out_specs=[
            pl.BlockSpec((BU, H), lambda i: (i, 0)),
            pl.BlockSpec((BU, H), lambda i: (i, 0)),
        ],
        out_shape=[
            jax.ShapeDtypeStruct((NPAD, H), jnp.float32),
            jax.ShapeDtypeStruct((NPAD, H), jnp.float32),
        ],
    )(xc, xv, w, b)


def _update_body(hc, hv, mp, wc, wv, bc, bv, hc_out, hv_out):
    mc = mp[0]
    mv = mp[1]
    wcf = wc[...]
    wvf = wv[...]
    hc_out[...] = jnp.maximum(
        jnp.dot(hc[...], wcf[:H], preferred_element_type=jnp.float32)
        + jnp.dot(mc, wcf[H:], preferred_element_type=jnp.float32) + bc[...], 0.0)
    hv_out[...] = jnp.maximum(
        jnp.dot(hv[...], wvf[:H], preferred_element_type=jnp.float32)
        + jnp.dot(mv, wvf[H:], preferred_element_type=jnp.float32) + bv[...], 0.0)


def _tc_update(hc, hv, mpart, wc, wv, bc, bv):
    grid = (NPAD // BU,)
    return pl.pallas_call(
        _update_body,
        grid=grid,
        in_specs=[
            pl.BlockSpec((BU, H), lambda i: (i, 0)),
            pl.BlockSpec((BU, H), lambda i: (i, 0)),
            pl.BlockSpec((2, BU, H), lambda i: (0, i, 0)),
            pl.BlockSpec((2 * H, H), lambda i: (0, 0)),
            pl.BlockSpec((2 * H, H), lambda i: (0, 0)),
            pl.BlockSpec((1, H), lambda i: (0, 0)),
            pl.BlockSpec((1, H), lambda i: (0, 0)),
        ],
        out_specs=[
            pl.BlockSpec((BU, H), lambda i: (i, 0)),
            pl.BlockSpec((BU, H), lambda i: (i, 0)),
        ],
        out_shape=[
            jax.ShapeDtypeStruct((NPAD, H), jnp.float32),
            jax.ShapeDtypeStruct((NPAD, H), jnp.float32),
        ],
    )(hc, hv, mpart, wc, wv, bc, bv)


def _heads_body(hc, hv, wm1, bm1, wm2, bm2, wl1, bl1, wl2, bl2, ec, ev,
                mean_c, mean_v, lv_c, lv_v, zc, zv):
    wm1f, wm2f, wl1f, wl2f = wm1[...], wm2[...], wl1[...], wl2[...]
    bm1f, bm2f, bl1f, bl2f = bm1[...], bm2[...], bl1[...], bl2[...]

    def head(h, w1, b1, w2, b2):
        t = jnp.dot(h, w1, preferred_element_type=jnp.float32) + b1
        return jnp.dot(t, w2, preferred_element_type=jnp.float32) + b2

    hcf = hc[...]
    hvf = hv[...]
    mc = head(hcf, wm1f, bm1f, wm2f, bm2f)
    mv = head(hvf, wm1f, bm1f, wm2f, bm2f)
    lc = head(hcf, wl1f, bl1f, wl2f, bl2f)
    lv = head(hvf, wl1f, bl1f, wl2f, bl2f)
    mean_c[...] = mc
    mean_v[...] = mv
    lv_c[...] = lc
    lv_v[...] = lv
    zc[...] = mc + ec[...] * jnp.exp(0.5 * lc)
    zv[...] = mv + ev[...] * jnp.exp(0.5 * lv)


def _tc_heads(hc, hv, wm1, bm1, wm2, bm2, wl1, bl1, wl2, bl2, ec, ev):
    grid = (NPAD // BU,)
    blk = lambda r, c: pl.BlockSpec((r, c), lambda i: (i, 0))
    full = lambda r, c: pl.BlockSpec((r, c), lambda i: (0, 0))
    return pl.pallas_call(
        _heads_body,
        grid=grid,
        in_specs=[
            blk(BU, H), blk(BU, H),
            full(H, 32), full(1, 32), full(32, L), full(1, L),
            full(H, 32), full(1, 32), full(32, L), full(1, L),
            blk(BU, L), blk(BU, L),
        ],
        out_specs=[blk(BU, L)] * 6,
        out_shape=[jax.ShapeDtypeStruct((NPAD, L), jnp.float32)] * 6,
    )(hc, hv, wm1, bm1, wm2, bm2, wl1, bl1, wl2, bl2, ec, ev)


BD = 200


def _decoder_body(zv, zc, out):
    logits = jax.lax.dot_general(zv[...], zc[...],
                                 (((1,), (1,)), ((), ())),
                                 preferred_element_type=jnp.float32)
    out[...] = jax.nn.sigmoid(logits)


def _tc_decoder(zv, zc):
    grid = (N // BD,)
    return pl.pallas_call(
        _decoder_body,
        grid=grid,
        in_specs=[
            pl.BlockSpec((BD, L), lambda i: (i, 0)),
            pl.BlockSpec((N, L), lambda i: (0, 0)),
        ],
        out_specs=pl.BlockSpec((BD, N), lambda i: (i, 0)),
        out_shape=jax.ShapeDtypeStruct((N, N), jnp.float32),
    )(zv, zc)


def kernel(x_v, x_c, edge_index, W_in, b_in, Wu_c, bu_c, Wu_v, bu_v,
           Whm1, bhm1, Whm2, bhm2, Whl1, bhl1, Whl2, bhl2):
    f32 = jnp.float32
    pad_rows = NPAD - N

    xv_p = jnp.pad(x_v, ((0, pad_rows), (0, 0)))
    xc_p = jnp.pad(x_c, ((0, pad_rows), (0, 0)))

    pad_e = E_PAD - E
    pad_idx = (N + (jnp.arange(pad_e, dtype=jnp.int32) % pad_rows)).astype(jnp.int32)
    src_p = jnp.concatenate([edge_index[0].astype(jnp.int32), pad_idx]).reshape(-1, CHUNK)
    dst_p = jnp.concatenate([edge_index[1].astype(jnp.int32), pad_idx]).reshape(-1, CHUNK)
    idxp0 = jnp.stack([src_p, dst_p], axis=1)
    idxp1 = jnp.stack([dst_p, src_p], axis=1)

    zeros_init = jnp.zeros((RPT, H), f32)

    b_in2 = b_in.reshape(1, H)
    bu_c2 = bu_c.reshape(1, H)
    bu_v2 = bu_v.reshape(1, H)

    h_c, h_v = _tc_input(xc_p, xv_p, W_in, b_in2)
    for _ in range(GNN_STEPS):
        mpart = _sc_messages(h_c, h_v, idxp0, idxp1, zeros_init)
        h_c, h_v = _tc_update(h_c, h_v, mpart, Wu_c, Wu_v, bu_c2, bu_v2)

    k1, k2 = jax.random.split(jax.random.key(42))
    eps_v = jax.random.normal(k1, (N, L), dtype=f32)
    eps_c = jax.random.normal(k2, (N, L), dtype=f32)
    ev_p = jnp.pad(eps_v, ((0, pad_rows), (0, 0)))
    ec_p = jnp.pad(eps_c, ((0, pad_rows), (0, 0)))

    mean_c, mean_v, lv_c, lv_v, z_c, z_v = _tc_heads(
        h_c, h_v, Whm1, bhm1.reshape(1, 32), Whm2, bhm2.reshape(1, L),
        Whl1, bhl1.reshape(1, 32), Whl2, bhl2.reshape(1, L), ec_p, ev_p)

    adj = _tc_decoder(z_v[:N], z_c[:N])

    return (adj,
            (mean_v[:N], mean_c[:N]),
            (lv_v[:N], lv_c[:N]))


# speedup vs baseline: 10.4395x; 1.0152x over previous
import functools

import jax
import jax.numpy as jnp
from jax import lax
from jax.experimental import pallas as pl
from jax.experimental.pallas import tpu as pltpu
from jax.experimental.pallas import tpu_sc as plsc

N = 5000
NPAD = 5120
H = 128
D = 128
L = 64
E = 320000
NC, NS = 2, 16
NW = NC * NS
CHUNK = 128
E_PAD = 327680
EPT = E_PAD // NS
CPT = EPT // CHUNK
RPT = NPAD // NS
GNN_STEPS = 3

_SC_MESH = plsc.VectorSubcoreMesh(core_axis_name="c", subcore_axis_name="s")


RING = 5
NT = CPT // RING


def _sc_messages_body(hc, hv, idxp0, idxp1, zeros, out,
                      idxb, rows, acc,
                      semi0, semi1,
                      semg0, semg1, semg2, semg3, semg4,
                      sems0, sems1, sems2, sems3, sems4):
    cid = lax.axis_index("c")
    sid = lax.axis_index("s")
    rbase = sid * RPT

    semi = (semi0, semi1)
    semg = (semg0, semg1, semg2, semg3, semg4)
    sems = (sems0, sems1, sems2, sems3, sems4)

    def run_direction(table, idxp, d):
        pltpu.sync_copy(zeros, acc.at[pl.ds(rbase, RPT)])
        plsc.subcore_barrier()
        cbase = sid * CPT

        def issue_idx(t, h):
            for j in range(RING):
                g = cbase + t * RING + j
                pltpu.async_copy(idxp.at[g], idxb.at[h, j], semi[h])

        def wait_idx(h):
            for j in range(RING):
                pltpu.make_async_copy(idxp.at[0], idxb.at[h, j], semi[h]).wait()

        def drain_scatters():
            for j in range(RING):
                pltpu.make_async_copy(table.at[pl.ds(0, CHUNK)], rows.at[j],
                                      sems[j]).wait()

        issue_idx(0, 0)

        def super_body(t, carry):
            for h in range(2):
                @pl.when(lax.rem(t, 2) == h)
                def _():
                    wait_idx(h)
                    for j in range(RING):
                        @pl.when(t > 0)
                        def _():
                            pltpu.make_async_copy(table.at[pl.ds(0, CHUNK)],
                                                  rows.at[j], sems[j]).wait()
                        pltpu.async_copy(table.at[idxb.at[h, j, 0]], rows.at[j],
                                         semg[j])
                    @pl.when(t + 1 < NT)
                    def _():
                        issue_idx(t + 1, 1 - h)
                    for j in range(RING):
                        pltpu.make_async_copy(table.at[pl.ds(0, CHUNK)],
                                              rows.at[j], semg[j]).wait()
                        pltpu.async_copy(rows.at[j], acc.at[idxb.at[h, j, 1]],
                                         sems[j], add=True)
            return carry

        lax.fori_loop(0, NT, super_body, 0)
        drain_scatters()
        plsc.subcore_barrier()
        pltpu.sync_copy(acc.at[pl.ds(rbase, RPT)], out.at[d, pl.ds(rbase, RPT)])

    @pl.when(cid == 0)
    def _():
        run_direction(hv, idxp0, 0)

    @pl.when(cid == 1)
    def _():
        run_direction(hc, idxp1, 1)


_sc_messages = functools.partial(
    pl.kernel,
    out_type=jax.ShapeDtypeStruct((2, NPAD, H), jnp.float32),
    mesh=_SC_MESH,
    scratch_types=[
        pltpu.VMEM((2, RING, 2, CHUNK), jnp.int32),
        pltpu.VMEM((RING, CHUNK, H), jnp.float32),
        pltpu.VMEM_SHARED((NPAD, H), jnp.float32),
    ] + [pltpu.SemaphoreType.DMA] * 12,
)(_sc_messages_body)


BU = 640


def _input_body(xc, xv, w, b, hc_out, hv_out):
    wf = w[...]
    bf = b[...]
    hc_out[...] = jnp.maximum(jnp.dot(xc[...], wf,
                                      preferred_element_type=jnp.float32) + bf, 0.0)
    hv_out[...] = jnp.maximum(jnp.dot(xv[...], wf,
                                      preferred_element_type=jnp.float32) + bf, 0.0)


def _tc_input(xc, xv, w, b):
    grid = (NPAD // BU,)
    return pl.pallas_call(
        _input_body,
        grid=grid,
        in_specs=[
            pl.BlockSpec((BU, D), lambda i: (i, 0)),
            pl.BlockSpec((BU, D), lambda i: (i, 0)),
            pl.BlockSpec((D, H), lambda i: (0, 0)),
            pl.BlockSpec((1, H), lambda i: (0, 0)),
        ],
        out_specs=[
            pl.BlockSpec((BU, H), lambda i: (i, 0)),
            pl.BlockSpec((BU, H), lambda i: (i, 0)),
        ],
        out_shape=[
            jax.ShapeDtypeStruct((NPAD, H), jnp.float32),
            jax.ShapeDtypeStruct((NPAD, H), jnp.float32),
        ],
    )(xc, xv, w, b)


def _update_body(hc, hv, mp, wc, wv, bc, bv, hc_out, hv_out):
    mc = mp[0]
    mv = mp[1]
    wcf = wc[...]
    wvf = wv[...]
    hc_out[...] = jnp.maximum(
        jnp.dot(hc[...], wcf[:H], preferred_element_type=jnp.float32)
        + jnp.dot(mc, wcf[H:], preferred_element_type=jnp.float32) + bc[...], 0.0)
    hv_out[...] = jnp.maximum(
        jnp.dot(hv[...], wvf[:H], preferred_element_type=jnp.float32)
        + jnp.dot(mv, wvf[H:], preferred_element_type=jnp.float32) + bv[...], 0.0)


def _tc_update(hc, hv, mpart, wc, wv, bc, bv):
    grid = (NPAD // BU,)
    return pl.pallas_call(
        _update_body,
        grid=grid,
        in_specs=[
            pl.BlockSpec((BU, H), lambda i: (i, 0)),
            pl.BlockSpec((BU, H), lambda i: (i, 0)),
            pl.BlockSpec((2, BU, H), lambda i: (0, i, 0)),
            pl.BlockSpec((2 * H, H), lambda i: (0, 0)),
            pl.BlockSpec((2 * H, H), lambda i: (0, 0)),
            pl.BlockSpec((1, H), lambda i: (0, 0)),
            pl.BlockSpec((1, H), lambda i: (0, 0)),
        ],
        out_specs=[
            pl.BlockSpec((BU, H), lambda i: (i, 0)),
            pl.BlockSpec((BU, H), lambda i: (i, 0)),
        ],
        out_shape=[
            jax.ShapeDtypeStruct((NPAD, H), jnp.float32),
            jax.ShapeDtypeStruct((NPAD, H), jnp.float32),
        ],
    )(hc, hv, mpart, wc, wv, bc, bv)


BD = 200


def _headdec_body(hc, hv, wm1, bm1, wm2, bm2, wl1, bl1, wl2, bl2, ec, ev,
                  mean_v, mean_c, lv_v, lv_c, adj, zvs, zcs):
    i = pl.program_id(0)

    @pl.when(i == 0)
    def _():
        wm1f, wm2f, wl1f, wl2f = wm1[...], wm2[...], wl1[...], wl2[...]
        bm1f, bm2f, bl1f, bl2f = bm1[...], bm2[...], bl1[...], bl2[...]

        def head(h, w1, b1, w2, b2):
            t = jnp.dot(h, w1, preferred_element_type=jnp.float32) + b1
            return jnp.dot(t, w2, preferred_element_type=jnp.float32) + b2

        hcf = hc[...][:N]
        hvf = hv[...][:N]
        mc = head(hcf, wm1f, bm1f, wm2f, bm2f)
        mv = head(hvf, wm1f, bm1f, wm2f, bm2f)
        lc = head(hcf, wl1f, bl1f, wl2f, bl2f)
        lv = head(hvf, wl1f, bl1f, wl2f, bl2f)
        mean_c[...] = mc
        mean_v[...] = mv
        lv_c[...] = lc
        lv_v[...] = lv
        zcs[...] = (mc + ec[...] * jnp.exp(0.5 * lc)).astype(jnp.bfloat16)
        zvs[...] = (mv + ev[...] * jnp.exp(0.5 * lv)).astype(jnp.bfloat16)

    logits = jax.lax.dot_general(zvs[pl.ds(i * BD, BD), :], zcs[...],
                                 (((1,), (1,)), ((), ())),
                                 preferred_element_type=jnp.float32)
    adj[...] = jax.nn.sigmoid(logits)


def _tc_headdec(hc, hv, wm1, bm1, wm2, bm2, wl1, bl1, wl2, bl2, ec, ev):
    grid = (N // BD,)
    full = lambda r, c: pl.BlockSpec((r, c), lambda i: (0, 0))
    return pl.pallas_call(
        _headdec_body,
        grid=grid,
        in_specs=[
            full(NPAD, H), full(NPAD, H),
            full(H, 32), full(1, 32), full(32, L), full(1, L),
            full(H, 32), full(1, 32), full(32, L), full(1, L),
            full(N, L), full(N, L),
        ],
        out_specs=[full(N, L)] * 4 + [pl.BlockSpec((BD, N), lambda i: (i, 0))],
        out_shape=[jax.ShapeDtypeStruct((N, L), jnp.float32)] * 4
        + [jax.ShapeDtypeStruct((N, N), jnp.float32)],
        scratch_shapes=[
            pltpu.VMEM((N, L), jnp.bfloat16),
            pltpu.VMEM((N, L), jnp.bfloat16),
        ],
    )(hc, hv, wm1, bm1, wm2, bm2, wl1, bl1, wl2, bl2, ec, ev)


def kernel(x_v, x_c, edge_index, W_in, b_in, Wu_c, bu_c, Wu_v, bu_v,
           Whm1, bhm1, Whm2, bhm2, Whl1, bhl1, Whl2, bhl2):
    f32 = jnp.float32
    pad_rows = NPAD - N

    xv_p = jnp.pad(x_v, ((0, pad_rows), (0, 0)))
    xc_p = jnp.pad(x_c, ((0, pad_rows), (0, 0)))

    pad_e = E_PAD - E
    pad_idx = (N + (jnp.arange(pad_e, dtype=jnp.int32) % pad_rows)).astype(jnp.int32)
    src_p = jnp.concatenate([edge_index[0].astype(jnp.int32), pad_idx]).reshape(-1, CHUNK)
    dst_p = jnp.concatenate([edge_index[1].astype(jnp.int32), pad_idx]).reshape(-1, CHUNK)
    idxp0 = jnp.stack([src_p, dst_p], axis=1)
    idxp1 = jnp.stack([dst_p, src_p], axis=1)

    zeros_init = jnp.zeros((RPT, H), f32)

    b_in2 = b_in.reshape(1, H)
    bu_c2 = bu_c.reshape(1, H)
    bu_v2 = bu_v.reshape(1, H)

    h_c, h_v = _tc_input(xc_p, xv_p, W_in, b_in2)
    for _ in range(GNN_STEPS):
        mpart = _sc_messages(h_c, h_v, idxp0, idxp1, zeros_init)
        h_c, h_v = _tc_update(h_c, h_v, mpart, Wu_c, Wu_v, bu_c2, bu_v2)

    k1, k2 = jax.random.split(jax.random.key(42))
    eps_v = jax.random.normal(k1, (N, L), dtype=f32)
    eps_c = jax.random.normal(k2, (N, L), dtype=f32)

    mean_v, mean_c, lv_v, lv_c, adj = _tc_headdec(
        h_c, h_v, Whm1, bhm1.reshape(1, 32), Whm2, bhm2.reshape(1, L),
        Whl1, bhl1.reshape(1, 32), Whl2, bhl2.reshape(1, L), eps_c, eps_v)

    return (adj, (mean_v, mean_c), (lv_v, lv_c))
